# baseline (speedup 1.0000x reference)
import functools

import jax
import jax.numpy as jnp
from jax import lax
from jax.experimental import pallas as pl
from jax.experimental.pallas import tpu as pltpu
from jax.experimental.pallas import tpu_sc as plsc

_B = 16384
_C = 1000
_NC = 2
_NS = 16
_NW = _NC * _NS
_ROWS_PER_W = _B // _NW
_CHUNK = 64
_NCHUNKS = _ROWS_PER_W // _CHUNK
_BUF = _CHUNK * _C


def _onehot_body(ids_hbm, out_hbm, buf_v, ids_v):
    c = lax.axis_index("c")
    s = lax.axis_index("s")
    wid = c * _NS + s
    row0 = wid * _ROWS_PER_W

    pltpu.sync_copy(ids_hbm.at[pl.ds(row0 * 1, _ROWS_PER_W)], ids_v)

    zeros16 = jnp.zeros((16,), jnp.float32)
    ones16 = jnp.ones((16,), jnp.float32)
    iota16 = lax.iota(jnp.int32, 16)

    def zero_body(i, carry):
        base = i * 128
        for u in range(8):
            buf_v[pl.ds(base + u * 16, 16)] = zeros16
        return carry

    lax.fori_loop(0, _BUF // 128, zero_body, 0)

    def flat_idx(k, g):
        ids16 = ids_v[pl.ds(k * _CHUNK + g * 16, 16)]
        in_vocab = (ids16 >= 0) & (ids16 < _C)
        mapped = jnp.where(in_vocab, ids16, _C - 1)
        return (g * 16 + iota16) * _C + mapped

    for k in range(_NCHUNKS):
        for g in range(_CHUNK // 16):
            plsc.store_scatter(buf_v, [flat_idx(k, g)], ones16)
        pltpu.sync_copy(
            buf_v, out_hbm.at[pl.ds((row0 + k * _CHUNK) * _C, _BUF)]
        )
        for g in range(_CHUNK // 16):
            plsc.store_scatter(buf_v, [flat_idx(k, g)], zeros16)


def kernel(user_ids):
    ids = user_ids.astype(jnp.int32)
    mesh = plsc.VectorSubcoreMesh(core_axis_name="c", subcore_axis_name="s")
    run = functools.partial(
        pl.kernel,
        mesh=mesh,
        out_type=jax.ShapeDtypeStruct((_B * _C,), jnp.float32),
        scratch_types=[
            pltpu.VMEM((_BUF,), jnp.float32),
            pltpu.VMEM((_ROWS_PER_W,), jnp.int32),
        ],
        compiler_params=pltpu.CompilerParams(needs_layout_passes=False),
    )(_onehot_body)
    out = run(ids)
    return out.reshape(_B, _C)
